# baseline (speedup 1.0000x reference)
import functools

import jax
import jax.numpy as jnp
from jax import lax
from jax.experimental import pallas as pl
from jax.experimental.pallas import tpu as pltpu
from jax.experimental.pallas import tpu_sc as plsc

_NC = 2
_NS = 16
_NW = _NC * _NS
_CH = 64
_L = 16


def _seg_of(ends_sc, row):
    cnt = jnp.int32(0)
    for e in ends_sc:
        cnt = cnt + jnp.where(e <= row, 1, 0).astype(jnp.int32)
    return cnt


def _sc_pool(flat, lengths):
    total, d = flat.shape
    nseg = lengths.shape[0]
    rows_w = total // _NW
    nch = rows_w // _CH
    nlane = d // _L
    mesh = plsc.VectorSubcoreMesh(core_axis_name="c", subcore_axis_name="s")

    @functools.partial(
        pl.kernel,
        mesh=mesh,
        out_type=jax.ShapeDtypeStruct((_NW, nseg, d), jnp.float32),
        scratch_types=[
            pltpu.VMEM((_CH, d), jnp.float32),
            pltpu.VMEM((_CH, d), jnp.float32),
            pltpu.VMEM((nseg,), jnp.int32),
            pltpu.VMEM((nseg, d), jnp.float32),
            pltpu.SemaphoreType.DMA,
            pltpu.SemaphoreType.DMA,
        ],
    )
    def pool(flat_hbm, len_hbm, out_hbm, buf0, buf1, len_v, acc, sem0, sem1):
        wid = lax.axis_index("s") * _NC + lax.axis_index("c")
        base = wid * rows_w

        pltpu.sync_copy(len_hbm, len_v)

        lv = len_v[...]
        ends_sc = []
        run = jnp.int32(0)
        for k in range(nseg):
            run = run + lv[k]
            ends_sc.append(run)

        def zero_body(i, _):
            acc[i // nlane, pl.ds((i % nlane) * _L, _L)] = jnp.zeros(
                (_L,), jnp.float32
            )
            return 0

        lax.fori_loop(0, nseg * nlane, zero_body, 0)

        bufs = (buf0, buf1)
        sems = (sem0, sem1)

        pltpu.async_copy(flat_hbm.at[pl.ds(base, _CH)], buf0, sem0)
        pltpu.async_copy(flat_hbm.at[pl.ds(base + _CH, _CH)], buf1, sem1)

        def process_chunk(i, buf):
            r0 = base + i * _CH
            seg_first = _seg_of(ends_sc, r0)
            seg_last = _seg_of(ends_sc, r0 + _CH - 1)

            def fast(seg0):
                def row_body(r, _):
                    for j in range(nlane):
                        plsc.addupdate(
                            acc.at[seg0, pl.ds(j * _L, _L)],
                            buf[r, pl.ds(j * _L, _L)],
                        )
                    return 0

                lax.fori_loop(0, _CH, row_body, 0)
                return 0

            def slow(_seg0):
                def row_body(r, _):
                    seg = _seg_of(ends_sc, r0 + r)
                    for j in range(nlane):
                        plsc.addupdate(
                            acc.at[seg, pl.ds(j * _L, _L)],
                            buf[r, pl.ds(j * _L, _L)],
                        )
                    return 0

                lax.fori_loop(0, _CH, row_body, 0)
                return 0

            lax.cond(seg_first == seg_last, fast, slow, seg_first)

        def group_body(g, _):
            for bsel in range(2):
                i = 2 * g + bsel
                pltpu.make_async_copy(
                    flat_hbm.at[pl.ds(base + i * _CH, _CH)],
                    bufs[bsel],
                    sems[bsel],
                ).wait()
                process_chunk(i, bufs[bsel])

                @pl.when(i + 2 < nch)
                def _refill():
                    pltpu.async_copy(
                        flat_hbm.at[pl.ds(base + (i + 2) * _CH, _CH)],
                        bufs[bsel],
                        sems[bsel],
                    )

            return 0

        lax.fori_loop(0, nch // 2, group_body, 0)

        pltpu.sync_copy(acc, out_hbm.at[wid])

    return pool(flat, lengths)


def _head_kernel(part_ref, len_ref, w_ref, bias_ref, out_ref):
    pooled = jnp.sum(part_ref[...], axis=0)
    inv = 1.0 / jnp.maximum(len_ref[...], 1).astype(jnp.float32)
    out_ref[...] = (
        jax.lax.dot_general(
            pooled * inv,
            w_ref[...],
            (((1,), (0,)), ((), ())),
            preferred_element_type=jnp.float32,
        )
        + bias_ref[...]
    )


def kernel(flat, past_lengths, W, b):
    total, d = flat.shape
    nseg = past_lengths.shape[0]
    lengths = past_lengths.astype(jnp.int32)
    partials = _sc_pool(flat, lengths)
    return pl.pallas_call(
        _head_kernel,
        in_specs=[
            pl.BlockSpec((_NW, nseg, d), lambda: (0, 0, 0)),
            pl.BlockSpec((nseg, 1), lambda: (0, 0)),
            pl.BlockSpec((d, d), lambda: (0, 0)),
            pl.BlockSpec((1, d), lambda: (0, 0)),
        ],
        out_specs=pl.BlockSpec((nseg, d), lambda: (0, 0)),
        out_shape=jax.ShapeDtypeStruct((nseg, d), jnp.float32),
    )(partials, lengths.reshape(nseg, 1), W, b.reshape(1, d))
